# Initial kernel scaffold; baseline (speedup 1.0000x reference)
import jax
import jax.numpy as jnp
from jax.experimental import pallas as pl


def kernel(x, edge_index, batch, W1, b1, g1, be1, W2, b2, g2, be2, W3, b3, g3, be3, Wl, bl):
    raise NotImplementedError("write your pallas kernel here")


# speedup vs baseline: 15.5253x; 15.5253x over previous
import functools

import jax
import jax.numpy as jnp
from jax import lax
from jax.experimental import pallas as pl
from jax.experimental.pallas import tpu as pltpu
from jax.experimental.pallas import tpu_sc as plsc

N = 50000
E = 800000
H = 64
G = 64

NC = 2
NS = 16
L = 16

EP = 802816
EROWS = EP // 128
BLK = 512

NPS = 51200
HH = H // NC
SUB_S = NPS // NS
SUB_F = NPS // NS

_mesh = plsc.VectorSubcoreMesh(core_axis_name="c", subcore_axis_name="s")
_sc_params = pltpu.CompilerParams(
    needs_layout_passes=False, use_tc_tiling_on_sc=False)
f32 = jnp.float32
i32 = jnp.int32


def _deg_body(dst_hbm, zeros_hbm, out_hbm, dst_v, ones_v, acc_sh):
    c = lax.axis_index("c")
    s = lax.axis_index("s")
    pltpu.sync_copy(zeros_hbm, acc_sh.at[pl.ds(s * SUB_S, SUB_S)])
    for k in range(8):
        ones_v[pl.ds(k * L, L)] = jnp.full((L,), 1.0, f32)
    plsc.subcore_barrier()
    w = c * NS + s
    base = w * (EROWS // (NC * NS))

    def body(i, _):
        row0 = base + i * 4
        pltpu.sync_copy(dst_hbm.at[pl.ds(row0, 4)], dst_v)
        for j in range(4):
            pltpu.sync_copy(ones_v, acc_sh.at[dst_v.at[j]], add=True)
        return 0

    lax.fori_loop(0, EROWS // (NC * NS) // 4, body, 0)
    plsc.subcore_barrier()
    pltpu.sync_copy(acc_sh.at[pl.ds(s * SUB_S, SUB_S)],
                    out_hbm.at[c, pl.ds(s * SUB_S, SUB_S)])


_deg_kernel = functools.partial(
    pl.kernel, _deg_body, mesh=_mesh, compiler_params=_sc_params,
    out_type=jax.ShapeDtypeStruct((NC, NPS), f32),
    scratch_types=[
        pltpu.VMEM((4, 128), i32),
        pltpu.VMEM((128,), f32),
        pltpu.VMEM_SHARED((NPS,), f32),
    ],
)()


def _sagg_body(src_hbm, dst_hbm, tab_hbm, zeros_hbm, out_hbm,
               src_v, dst_v, vals_v, tab_v, acc_sh):
    c = lax.axis_index("c")
    s = lax.axis_index("s")
    pltpu.sync_copy(zeros_hbm, acc_sh.at[pl.ds(s * SUB_S, SUB_S)])
    pltpu.sync_copy(tab_hbm, tab_v)
    plsc.subcore_barrier()
    w = c * NS + s
    base = w * (EROWS // (NC * NS))

    def body(i, _):
        row0 = base + i * 4
        pltpu.sync_copy(src_hbm.at[pl.ds(row0, 4)], src_v)
        pltpu.sync_copy(dst_hbm.at[pl.ds(row0, 4)], dst_v)
        for j in range(4):
            for k in range(8):
                sidx = src_v[j, pl.ds(k * L, L)]
                vals_v[j, pl.ds(k * L, L)] = plsc.load_gather(tab_v, [sidx])
        for j in range(4):
            pltpu.sync_copy(vals_v.at[j], acc_sh.at[dst_v.at[j]], add=True)
        return 0

    lax.fori_loop(0, EROWS // (NC * NS) // 4, body, 0)
    plsc.subcore_barrier()
    pltpu.sync_copy(acc_sh.at[pl.ds(s * SUB_S, SUB_S)],
                    out_hbm.at[c, pl.ds(s * SUB_S, SUB_S)])


_sagg_kernel = functools.partial(
    pl.kernel, _sagg_body, mesh=_mesh, compiler_params=_sc_params,
    out_type=jax.ShapeDtypeStruct((NC, NPS), f32),
    scratch_types=[
        pltpu.VMEM((4, 128), i32),
        pltpu.VMEM((4, 128), i32),
        pltpu.VMEM((4, 128), f32),
        pltpu.VMEM((NPS,), f32),
        pltpu.VMEM_SHARED((NPS,), f32),
    ],
)()


def _ragg_body(y_hbm, src_hbm, dst_hbm, zeros_hbm, out_hbm,
               src_v, srca_v, dst_v, rows_v, acc_sh, sem):
    c = lax.axis_index("c")
    s = lax.axis_index("s")
    pltpu.sync_copy(zeros_hbm, acc_sh.at[pl.ds(s * SUB_F, SUB_F)])
    plsc.subcore_barrier()
    cbase = c * N
    base = s * (EROWS // NS)

    def body(i, _):
        row0 = base + i * 4
        pltpu.sync_copy(src_hbm.at[pl.ds(row0, 4)], src_v)
        pltpu.sync_copy(dst_hbm.at[pl.ds(row0, 4)], dst_v)
        for j in range(4):
            for k in range(8):
                srca_v[j, pl.ds(k * L, L)] = src_v[j, pl.ds(k * L, L)] + cbase
        descs = [
            pltpu.async_copy(y_hbm.at[srca_v.at[j]],
                             rows_v.at[pl.ds(j * 128, 128)], sem)
            for j in range(4)
        ]
        for de in descs:
            de.wait()
        for j in range(4):
            pltpu.sync_copy(rows_v.at[pl.ds(j * 128, 128)],
                            acc_sh.at[dst_v.at[j]], add=True)
        return 0

    lax.fori_loop(0, EROWS // NS // 4, body, 0)
    plsc.subcore_barrier()
    pltpu.sync_copy(acc_sh.at[pl.ds(s * SUB_F, SUB_F)],
                    out_hbm.at[c, pl.ds(s * SUB_F, SUB_F)])


_ragg_kernel = functools.partial(
    pl.kernel, _ragg_body, mesh=_mesh, compiler_params=_sc_params,
    out_type=jax.ShapeDtypeStruct((NC, NPS, HH), f32),
    scratch_types=[
        pltpu.VMEM((4, 128), i32),
        pltpu.VMEM((4, 128), i32),
        pltpu.VMEM((4, 128), i32),
        pltpu.VMEM((BLK, HH), f32),
        pltpu.VMEM_SHARED((NPS, HH), f32),
        pltpu.SemaphoreType.DMA,
    ],
)()


def _t1_body(degp_ref, x_ref, dinv_ref, y1_ref):
    deg = degp_ref[0] + degp_ref[1] + 1.0
    dinv = lax.rsqrt(deg)
    dinv_ref[...] = dinv
    y1_ref[...] = dinv * x_ref[...]


def _t1(degp, xp):
    return pl.pallas_call(
        _t1_body,
        grid=(16,),
        in_specs=[
            pl.BlockSpec((2, 16, 200), lambda g: (0, g, 0)),
            pl.BlockSpec((16, 200), lambda g: (g, 0)),
        ],
        out_specs=[
            pl.BlockSpec((16, 200), lambda g: (g, 0)),
            pl.BlockSpec((16, 200), lambda g: (g, 0)),
        ],
        out_shape=[
            jax.ShapeDtypeStruct((256, 200), f32),
            jax.ShapeDtypeStruct((256, 200), f32),
        ],
    )(degp, xp)


def _t2a_body(aggp_ref, y1_ref, dinv_ref, s_ref, ssum_ref, ssq_ref):
    g = pl.program_id(0)
    sv = dinv_ref[...] * (aggp_ref[0] + aggp_ref[1] + y1_ref[...])
    s_ref[...] = sv
    rows = g * 16 + lax.broadcasted_iota(i32, (16, 200), 0)
    sm = jnp.where(rows < 250, sv, 0.0)

    @pl.when(g == 0)
    def _():
        ssum_ref[...] = jnp.zeros((1, 1), f32)
        ssq_ref[...] = jnp.zeros((1, 1), f32)

    ssum_ref[...] += jnp.sum(sm).reshape(1, 1)
    ssq_ref[...] += jnp.sum(sm * sm).reshape(1, 1)


def _t2a(aggp, y1, dinv):
    return pl.pallas_call(
        _t2a_body,
        grid=(16,),
        in_specs=[
            pl.BlockSpec((2, 16, 200), lambda g: (0, g, 0)),
            pl.BlockSpec((16, 200), lambda g: (g, 0)),
            pl.BlockSpec((16, 200), lambda g: (g, 0)),
        ],
        out_specs=[
            pl.BlockSpec((16, 200), lambda g: (g, 0)),
            pl.BlockSpec((1, 1), lambda g: (0, 0)),
            pl.BlockSpec((1, 1), lambda g: (0, 0)),
        ],
        out_shape=[
            jax.ShapeDtypeStruct((256, 200), f32),
            jax.ShapeDtypeStruct((1, 1), f32),
            jax.ShapeDtypeStruct((1, 1), f32),
        ],
    )(aggp, y1, dinv)


def _t2b_body(s_ref, dinv_ref, a_ref, d_ref, w_ref, y2_ref):
    h1 = jnp.maximum(s_ref[...] * a_ref[...] + d_ref[...], 0.0)
    y2 = dinv_ref[...] * jnp.dot(h1, w_ref[...], preferred_element_type=f32)
    y2_ref[0] = y2[:, :HH]
    y2_ref[1] = y2[:, HH:]


def _t2b(s_col, dinv_col, a1c, d1c, W2):
    return pl.pallas_call(
        _t2b_body,
        grid=(250,),
        in_specs=[
            pl.BlockSpec((200, 1), lambda g: (g, 0)),
            pl.BlockSpec((200, 1), lambda g: (g, 0)),
            pl.BlockSpec((1, H), lambda g: (0, 0)),
            pl.BlockSpec((1, H), lambda g: (0, 0)),
            pl.BlockSpec((H, H), lambda g: (0, 0)),
        ],
        out_specs=pl.BlockSpec((NC, 200, HH), lambda g: (0, g, 0)),
        out_shape=jax.ShapeDtypeStruct((NC, N, HH), f32),
    )(s_col, dinv_col, a1c, d1c, W2)


def _zfull(aggp_ref, y_ref, dinv_ref, b_ref):
    agg = jnp.concatenate([aggp_ref[0], aggp_ref[1]], axis=1)
    y = jnp.concatenate([y_ref[0], y_ref[1]], axis=1)
    return dinv_ref[...] * (agg + y) + b_ref[...]


def _stats_body(aggp_ref, y_ref, dinv_ref, b_ref, ssum_ref, ssq_ref):
    g = pl.program_id(0)
    z = _zfull(aggp_ref, y_ref, dinv_ref, b_ref)

    @pl.when(g == 0)
    def _():
        ssum_ref[...] = jnp.zeros((1, H), f32)
        ssq_ref[...] = jnp.zeros((1, H), f32)

    ssum_ref[...] += jnp.sum(z, axis=0, keepdims=True)
    ssq_ref[...] += jnp.sum(z * z, axis=0, keepdims=True)


def _tstats(aggp, y, dinv_col, b_row):
    return pl.pallas_call(
        _stats_body,
        grid=(250,),
        in_specs=[
            pl.BlockSpec((NC, 200, HH), lambda g: (0, g, 0)),
            pl.BlockSpec((NC, 200, HH), lambda g: (0, g, 0)),
            pl.BlockSpec((200, 1), lambda g: (g, 0)),
            pl.BlockSpec((1, H), lambda g: (0, 0)),
        ],
        out_specs=[
            pl.BlockSpec((1, H), lambda g: (0, 0)),
            pl.BlockSpec((1, H), lambda g: (0, 0)),
        ],
        out_shape=[
            jax.ShapeDtypeStruct((1, H), f32),
            jax.ShapeDtypeStruct((1, H), f32),
        ],
    )(aggp, y, dinv_col, b_row)


def _next_body(aggp_ref, y_ref, dinv_ref, b_ref, a_ref, d_ref, w_ref, out_ref):
    z = _zfull(aggp_ref, y_ref, dinv_ref, b_ref)
    h = jnp.maximum(z * a_ref[...] + d_ref[...], 0.0)
    out = dinv_ref[...] * jnp.dot(h, w_ref[...], preferred_element_type=f32)
    out_ref[0] = out[:, :HH]
    out_ref[1] = out[:, HH:]


def _tnext(aggp, y, dinv_col, b_row, a_row, d_row, W):
    return pl.pallas_call(
        _next_body,
        grid=(250,),
        in_specs=[
            pl.BlockSpec((NC, 200, HH), lambda g: (0, g, 0)),
            pl.BlockSpec((NC, 200, HH), lambda g: (0, g, 0)),
            pl.BlockSpec((200, 1), lambda g: (g, 0)),
            pl.BlockSpec((1, H), lambda g: (0, 0)),
            pl.BlockSpec((1, H), lambda g: (0, 0)),
            pl.BlockSpec((1, H), lambda g: (0, 0)),
            pl.BlockSpec((H, H), lambda g: (0, 0)),
        ],
        out_specs=pl.BlockSpec((NC, 200, HH), lambda g: (0, g, 0)),
        out_shape=jax.ShapeDtypeStruct((NC, N, HH), f32),
    )(aggp, y, dinv_col, b_row, a_row, d_row, W)


def _pool_body(aggp_ref, y_ref, dinv_ref, b_ref, a_ref, d_ref, batch_ref,
               pooled_ref, cnt_ref):
    g = pl.program_id(0)
    z = _zfull(aggp_ref, y_ref, dinv_ref, b_ref)
    h = jnp.maximum(z * a_ref[...] + d_ref[...], 0.0)
    gids = lax.broadcasted_iota(i32, (1, G), 1)
    oh = (batch_ref[...] == gids).astype(f32)

    @pl.when(g == 0)
    def _():
        pooled_ref[...] = jnp.zeros((G, H), f32)
        cnt_ref[...] = jnp.zeros((G, 1), f32)

    pooled_ref[...] += lax.dot_general(
        oh, h, (((0,), (0,)), ((), ())), preferred_element_type=f32)
    cnt_ref[...] += lax.dot_general(
        oh, jnp.ones((200, 1), f32), (((0,), (0,)), ((), ())),
        preferred_element_type=f32)


def _tpool(aggp, y, dinv_col, b_row, a_row, d_row, batch_col):
    return pl.pallas_call(
        _pool_body,
        grid=(250,),
        in_specs=[
            pl.BlockSpec((NC, 200, HH), lambda g: (0, g, 0)),
            pl.BlockSpec((NC, 200, HH), lambda g: (0, g, 0)),
            pl.BlockSpec((200, 1), lambda g: (g, 0)),
            pl.BlockSpec((1, H), lambda g: (0, 0)),
            pl.BlockSpec((1, H), lambda g: (0, 0)),
            pl.BlockSpec((1, H), lambda g: (0, 0)),
            pl.BlockSpec((200, 1), lambda g: (g, 0)),
        ],
        out_specs=[
            pl.BlockSpec((G, H), lambda g: (0, 0)),
            pl.BlockSpec((G, 1), lambda g: (0, 0)),
        ],
        out_shape=[
            jax.ShapeDtypeStruct((G, H), f32),
            jax.ShapeDtypeStruct((G, 1), f32),
        ],
    )(aggp, y, dinv_col, b_row, a_row, d_row, batch_col)


def _final_body(pooled_ref, cnt_ref, wl_ref, bl_ref, out_ref):
    mean = pooled_ref[...] / jnp.maximum(cnt_ref[...], 1.0)
    out_ref[...] = jnp.dot(
        mean, wl_ref[...], preferred_element_type=f32) + bl_ref[...]


def _tfinal(pooled, cnt, Wl, bl_row):
    return pl.pallas_call(
        _final_body,
        out_shape=jax.ShapeDtypeStruct((G, 2), f32),
    )(pooled, cnt, Wl, bl_row)


def kernel(x, edge_index, batch, W1, b1, g1, be1, W2, b2, g2, be2,
           W3, b3, g3, be3, Wl, bl):
    eps = 1e-5
    pad = EP - E
    srcp = jnp.concatenate(
        [edge_index[0], jnp.zeros((pad,), i32)]).reshape(EROWS, 128)
    dstp = jnp.concatenate(
        [edge_index[1], jnp.full((pad,), N, i32)]).reshape(EROWS, 128)
    zeros_s = jnp.zeros((SUB_S,), f32)
    zeros_f = jnp.zeros((SUB_F, HH), f32)
    xp = jnp.concatenate([x[:, 0], jnp.zeros((NPS - N,), f32)]).reshape(256, 200)

    degp = _deg_kernel(dstp, zeros_s).reshape(NC, 256, 200)
    dinv, y1 = _t1(degp, xp)

    aggp1 = _sagg_kernel(srcp, dstp, y1.reshape(NPS), zeros_s)
    aggp1 = aggp1.reshape(NC, 256, 200)
    s2d, ssum, ssq = _t2a(aggp1, y1, dinv)
    sbar = ssum[0, 0] / N
    var_s = ssq[0, 0] / N - sbar * sbar
    w1 = W1[0]
    a1c = (w1 * g1 / jnp.sqrt(var_s * w1 * w1 + eps)).reshape(1, H)
    d1c = be1.reshape(1, H) - sbar * a1c

    s_col = s2d.reshape(NPS, 1)[:N]
    dinv_col = dinv.reshape(NPS, 1)[:N]
    y2 = _t2b(s_col, dinv_col, a1c, d1c, W2)

    aggp2 = _ragg_kernel(y2.reshape(NC * N, HH), srcp, dstp, zeros_f)
    b2r = b2.reshape(1, H)
    ssum2, ssq2 = _tstats(aggp2, y2, dinv_col, b2r)
    mu2 = ssum2 / N
    var2 = ssq2 / N - mu2 * mu2
    a2c = g2.reshape(1, H) / jnp.sqrt(var2 + eps)
    d2c = be2.reshape(1, H) - mu2 * a2c
    y3 = _tnext(aggp2, y2, dinv_col, b2r, a2c, d2c, W3)

    aggp3 = _ragg_kernel(y3.reshape(NC * N, HH), srcp, dstp, zeros_f)
    b3r = b3.reshape(1, H)
    ssum3, ssq3 = _tstats(aggp3, y3, dinv_col, b3r)
    mu3 = ssum3 / N
    var3 = ssq3 / N - mu3 * mu3
    a3c = g3.reshape(1, H) / jnp.sqrt(var3 + eps)
    d3c = be3.reshape(1, H) - mu3 * a3c

    batch_col = batch.reshape(N, 1)
    pooled, cnt = _tpool(aggp3, y3, dinv_col, b3r, a3c, d3c, batch_col)
    return _tfinal(pooled, cnt, Wl, bl.reshape(1, 2))


# speedup vs baseline: 19.5578x; 1.2597x over previous
import functools

import jax
import jax.numpy as jnp
from jax import lax
from jax.experimental import pallas as pl
from jax.experimental.pallas import tpu as pltpu
from jax.experimental.pallas import tpu_sc as plsc

N = 50000
E = 800000
H = 64
G = 64

NC = 2
NS = 16
L = 16

EP = 802816
EROWS = EP // 128

NPS = 51200
HH = H // NC
SUB_S = NPS // NS

W_ROWS = EROWS // (NC * NS)
S_ROWS = EROWS // NS

NPF = 50400
SUB_F = NPF // NS

_mesh = plsc.VectorSubcoreMesh(core_axis_name="c", subcore_axis_name="s")
_sc_params = pltpu.CompilerParams(
    needs_layout_passes=False, use_tc_tiling_on_sc=False)
f32 = jnp.float32
i32 = jnp.int32


def _deg_body(pk_hbm, zeros_hbm, out_hbm, slab_v, dst_v, ones_v, acc_sh):
    c = lax.axis_index("c")
    s = lax.axis_index("s")
    pltpu.sync_copy(zeros_hbm, acc_sh.at[pl.ds(s * SUB_S, SUB_S)])
    for k in range(8):
        ones_v[pl.ds(k * L, L)] = jnp.full((L,), 1.0, f32)
    w = c * NS + s
    pltpu.sync_copy(pk_hbm.at[pl.ds(w * W_ROWS * 128, W_ROWS * 128)], slab_v)
    plsc.subcore_barrier()

    def body(i, _):
        for j in range(4):
            for k in range(8):
                off = i * 512 + j * 128 + k * L
                dst_v[j, pl.ds(k * L, L)] = jnp.bitwise_and(
                    slab_v[pl.ds(off, L)], 0xFFFF)
        for j in range(4):
            pltpu.sync_copy(ones_v, acc_sh.at[dst_v.at[j]], add=True)
        return 0

    lax.fori_loop(0, W_ROWS // 4, body, 0)
    plsc.subcore_barrier()
    pltpu.sync_copy(acc_sh.at[pl.ds(s * SUB_S, SUB_S)],
                    out_hbm.at[c, pl.ds(s * SUB_S, SUB_S)])


_deg_kernel = functools.partial(
    pl.kernel, _deg_body, mesh=_mesh, compiler_params=_sc_params,
    out_type=jax.ShapeDtypeStruct((NC, NPS), f32),
    scratch_types=[
        pltpu.VMEM((W_ROWS * 128,), i32),
        pltpu.VMEM((4, 128), i32),
        pltpu.VMEM((128,), f32),
        pltpu.VMEM_SHARED((NPS,), f32),
    ],
)()


def _sagg_body(pk_hbm, tab_hbm, zeros_hbm, out_hbm,
               slab_v, dst_v, vals_v, tab_v, acc_sh):
    c = lax.axis_index("c")
    s = lax.axis_index("s")
    pltpu.sync_copy(zeros_hbm, acc_sh.at[pl.ds(s * SUB_S, SUB_S)])
    pltpu.sync_copy(tab_hbm, tab_v)
    w = c * NS + s
    pltpu.sync_copy(pk_hbm.at[pl.ds(w * W_ROWS * 128, W_ROWS * 128)], slab_v)
    plsc.subcore_barrier()

    def body(i, _):
        for j in range(4):
            for k in range(8):
                off = i * 512 + j * 128 + k * L
                pk = slab_v[pl.ds(off, L)]
                sidx = lax.shift_right_logical(pk, 16)
                vals_v[j, pl.ds(k * L, L)] = plsc.load_gather(tab_v, [sidx])
                dst_v[j, pl.ds(k * L, L)] = jnp.bitwise_and(pk, 0xFFFF)
        for j in range(4):
            pltpu.sync_copy(vals_v.at[j], acc_sh.at[dst_v.at[j]], add=True)
        return 0

    lax.fori_loop(0, W_ROWS // 4, body, 0)
    plsc.subcore_barrier()
    pltpu.sync_copy(acc_sh.at[pl.ds(s * SUB_S, SUB_S)],
                    out_hbm.at[c, pl.ds(s * SUB_S, SUB_S)])


_sagg_kernel = functools.partial(
    pl.kernel, _sagg_body, mesh=_mesh, compiler_params=_sc_params,
    out_type=jax.ShapeDtypeStruct((NC, NPS), f32),
    scratch_types=[
        pltpu.VMEM((W_ROWS * 128,), i32),
        pltpu.VMEM((4, 128), i32),
        pltpu.VMEM((4, 128), f32),
        pltpu.VMEM((NPS,), f32),
        pltpu.VMEM_SHARED((NPS,), f32),
    ],
)()


def _ragg_body(y_hbm, pk_hbm, zeros_hbm, out_hbm,
               idx_v, srca_v, dst_v, rows_v, acc_sh, sem0, sem1, isem0, isem1):
    c = lax.axis_index("c")
    s = lax.axis_index("s")
    sems = (sem0, sem1)
    isems = (isem0, isem1)
    pltpu.sync_copy(zeros_hbm, acc_sh.at[pl.ds(s * SUB_F, SUB_F)])
    cbase = c * N
    base = s * S_ROWS
    plsc.subcore_barrier()

    def idx_desc(g, b):
        return pltpu.make_async_copy(
            pk_hbm.at[pl.ds(base + g * 2, 2)],
            idx_v.at[pl.ds(b * 2, 2)], isems[b])

    def unpack(b):
        for j in range(2):
            for k in range(8):
                pk = idx_v[b * 2 + j, pl.ds(k * L, L)]
                srca_v[2 * b + j, pl.ds(k * L, L)] = lax.shift_right_logical(
                    pk, 16) + cbase
                dst_v[2 * b + j, pl.ds(k * L, L)] = jnp.bitwise_and(pk, 0xFFFF)

    def descs(b):
        return [
            pltpu.make_async_copy(
                y_hbm.at[srca_v.at[2 * b + j]],
                rows_v.at[pl.ds(b * 256 + j * 128, 128)], sems[b])
            for j in range(2)
        ]

    def drain_scatter(b):
        for de in descs(b):
            de.wait()
        for j in range(2):
            pltpu.sync_copy(rows_v.at[pl.ds(b * 256 + j * 128, 128)],
                            acc_sh.at[dst_v.at[2 * b + j]], add=True)

    for b in range(2):
        idx_desc(b, b).start()
    for b in range(2):
        idx_desc(b, b).wait()
        unpack(b)
        for de in descs(b):
            de.start()
        idx_desc(b + 2, b).start()

    def body(i, _):
        for b in range(2):
            g = i * 2 + b
            drain_scatter(b)
            idx_desc(g + 2, b).wait()
            unpack(b)
            for de in descs(b):
                de.start()
            idx_desc(g + 4, b).start()
        return 0

    nblk = S_ROWS // 2
    lax.fori_loop(0, nblk // 2 - 2, body, 0)
    for b in range(2):
        g = nblk - 4 + b
        drain_scatter(b)
        idx_desc(g + 2, b).wait()
        unpack(b)
        for de in descs(b):
            de.start()
    for b in range(2):
        drain_scatter(b)

    plsc.subcore_barrier()
    pltpu.sync_copy(acc_sh.at[pl.ds(s * SUB_F, SUB_F)],
                    out_hbm.at[c, pl.ds(s * SUB_F, SUB_F)])


_ragg_kernel = functools.partial(
    pl.kernel, _ragg_body, mesh=_mesh, compiler_params=_sc_params,
    out_type=jax.ShapeDtypeStruct((NC, NPF, HH), f32),
    scratch_types=[
        pltpu.VMEM((4, 128), i32),
        pltpu.VMEM((4, 128), i32),
        pltpu.VMEM((4, 128), i32),
        pltpu.VMEM((512, HH), f32),
        pltpu.VMEM_SHARED((NPF, HH), f32),
        pltpu.SemaphoreType.DMA,
        pltpu.SemaphoreType.DMA,
        pltpu.SemaphoreType.DMA,
        pltpu.SemaphoreType.DMA,
    ],
)()


def _t1_body(degp_ref, x_ref, dinv_ref, y1_ref):
    deg = degp_ref[0] + degp_ref[1] + 1.0
    dinv = lax.rsqrt(deg)
    dinv_ref[...] = dinv
    y1_ref[...] = dinv * x_ref[...]


def _t1(degp, xp):
    return pl.pallas_call(
        _t1_body,
        grid=(16,),
        in_specs=[
            pl.BlockSpec((2, 16, 200), lambda g: (0, g, 0)),
            pl.BlockSpec((16, 200), lambda g: (g, 0)),
        ],
        out_specs=[
            pl.BlockSpec((16, 200), lambda g: (g, 0)),
            pl.BlockSpec((16, 200), lambda g: (g, 0)),
        ],
        out_shape=[
            jax.ShapeDtypeStruct((256, 200), f32),
            jax.ShapeDtypeStruct((256, 200), f32),
        ],
    )(degp, xp)


def _t2a_body(aggp_ref, y1_ref, dinv_ref, s_ref, ssum_ref, ssq_ref):
    g = pl.program_id(0)
    sv = dinv_ref[...] * (aggp_ref[0] + aggp_ref[1] + y1_ref[...])
    s_ref[...] = sv
    rows = g * 16 + lax.broadcasted_iota(i32, (16, 200), 0)
    sm = jnp.where(rows < 250, sv, 0.0)

    @pl.when(g == 0)
    def _():
        ssum_ref[...] = jnp.zeros((1, 1), f32)
        ssq_ref[...] = jnp.zeros((1, 1), f32)

    ssum_ref[...] += jnp.sum(sm).reshape(1, 1)
    ssq_ref[...] += jnp.sum(sm * sm).reshape(1, 1)


def _t2a(aggp, y1, dinv):
    return pl.pallas_call(
        _t2a_body,
        grid=(16,),
        in_specs=[
            pl.BlockSpec((2, 16, 200), lambda g: (0, g, 0)),
            pl.BlockSpec((16, 200), lambda g: (g, 0)),
            pl.BlockSpec((16, 200), lambda g: (g, 0)),
        ],
        out_specs=[
            pl.BlockSpec((16, 200), lambda g: (g, 0)),
            pl.BlockSpec((1, 1), lambda g: (0, 0)),
            pl.BlockSpec((1, 1), lambda g: (0, 0)),
        ],
        out_shape=[
            jax.ShapeDtypeStruct((256, 200), f32),
            jax.ShapeDtypeStruct((1, 1), f32),
            jax.ShapeDtypeStruct((1, 1), f32),
        ],
    )(aggp, y1, dinv)


def _t2b_body(s_ref, dinv_ref, a_ref, d_ref, w_ref, y2_ref):
    h1 = jnp.maximum(s_ref[...] * a_ref[...] + d_ref[...], 0.0)
    y2 = dinv_ref[...] * jnp.dot(h1, w_ref[...], preferred_element_type=f32)
    y2_ref[0] = y2[:, :HH]
    y2_ref[1] = y2[:, HH:]


def _t2b(s_col, dinv_col, a1c, d1c, W2):
    return pl.pallas_call(
        _t2b_body,
        grid=(250,),
        in_specs=[
            pl.BlockSpec((200, 1), lambda g: (g, 0)),
            pl.BlockSpec((200, 1), lambda g: (g, 0)),
            pl.BlockSpec((1, H), lambda g: (0, 0)),
            pl.BlockSpec((1, H), lambda g: (0, 0)),
            pl.BlockSpec((H, H), lambda g: (0, 0)),
        ],
        out_specs=pl.BlockSpec((NC, 200, HH), lambda g: (0, g, 0)),
        out_shape=jax.ShapeDtypeStruct((NC, N, HH), f32),
    )(s_col, dinv_col, a1c, d1c, W2)


def _zfull(aggp_ref, y_ref, dinv_ref, b_ref):
    agg = jnp.concatenate([aggp_ref[0], aggp_ref[1]], axis=1)
    y = jnp.concatenate([y_ref[0], y_ref[1]], axis=1)
    return dinv_ref[...] * (agg + y) + b_ref[...]


def _stats_body(aggp_ref, y_ref, dinv_ref, b_ref, ssum_ref, ssq_ref):
    g = pl.program_id(0)
    z = _zfull(aggp_ref, y_ref, dinv_ref, b_ref)

    @pl.when(g == 0)
    def _():
        ssum_ref[...] = jnp.zeros((1, H), f32)
        ssq_ref[...] = jnp.zeros((1, H), f32)

    ssum_ref[...] += jnp.sum(z, axis=0, keepdims=True)
    ssq_ref[...] += jnp.sum(z * z, axis=0, keepdims=True)


def _tstats(aggp, y, dinv_col, b_row):
    return pl.pallas_call(
        _stats_body,
        grid=(250,),
        in_specs=[
            pl.BlockSpec((NC, 200, HH), lambda g: (0, g, 0)),
            pl.BlockSpec((NC, 200, HH), lambda g: (0, g, 0)),
            pl.BlockSpec((200, 1), lambda g: (g, 0)),
            pl.BlockSpec((1, H), lambda g: (0, 0)),
        ],
        out_specs=[
            pl.BlockSpec((1, H), lambda g: (0, 0)),
            pl.BlockSpec((1, H), lambda g: (0, 0)),
        ],
        out_shape=[
            jax.ShapeDtypeStruct((1, H), f32),
            jax.ShapeDtypeStruct((1, H), f32),
        ],
    )(aggp, y, dinv_col, b_row)


def _next_body(aggp_ref, y_ref, dinv_ref, b_ref, a_ref, d_ref, w_ref, out_ref):
    z = _zfull(aggp_ref, y_ref, dinv_ref, b_ref)
    h = jnp.maximum(z * a_ref[...] + d_ref[...], 0.0)
    out = dinv_ref[...] * jnp.dot(h, w_ref[...], preferred_element_type=f32)
    out_ref[0] = out[:, :HH]
    out_ref[1] = out[:, HH:]


def _tnext(aggp, y, dinv_col, b_row, a_row, d_row, W):
    return pl.pallas_call(
        _next_body,
        grid=(250,),
        in_specs=[
            pl.BlockSpec((NC, 200, HH), lambda g: (0, g, 0)),
            pl.BlockSpec((NC, 200, HH), lambda g: (0, g, 0)),
            pl.BlockSpec((200, 1), lambda g: (g, 0)),
            pl.BlockSpec((1, H), lambda g: (0, 0)),
            pl.BlockSpec((1, H), lambda g: (0, 0)),
            pl.BlockSpec((1, H), lambda g: (0, 0)),
            pl.BlockSpec((H, H), lambda g: (0, 0)),
        ],
        out_specs=pl.BlockSpec((NC, 200, HH), lambda g: (0, g, 0)),
        out_shape=jax.ShapeDtypeStruct((NC, N, HH), f32),
    )(aggp, y, dinv_col, b_row, a_row, d_row, W)


def _pool_body(aggp_ref, y_ref, dinv_ref, b_ref, a_ref, d_ref, batch_ref,
               pooled_ref, cnt_ref):
    g = pl.program_id(0)
    z = _zfull(aggp_ref, y_ref, dinv_ref, b_ref)
    h = jnp.maximum(z * a_ref[...] + d_ref[...], 0.0)
    gids = lax.broadcasted_iota(i32, (1, G), 1)
    oh = (batch_ref[...] == gids).astype(f32)

    @pl.when(g == 0)
    def _():
        pooled_ref[...] = jnp.zeros((G, H), f32)
        cnt_ref[...] = jnp.zeros((G, 1), f32)

    pooled_ref[...] += lax.dot_general(
        oh, h, (((0,), (0,)), ((), ())), preferred_element_type=f32)
    cnt_ref[...] += lax.dot_general(
        oh, jnp.ones((200, 1), f32), (((0,), (0,)), ((), ())),
        preferred_element_type=f32)


def _tpool(aggp, y, dinv_col, b_row, a_row, d_row, batch_col):
    return pl.pallas_call(
        _pool_body,
        grid=(250,),
        in_specs=[
            pl.BlockSpec((NC, 200, HH), lambda g: (0, g, 0)),
            pl.BlockSpec((NC, 200, HH), lambda g: (0, g, 0)),
            pl.BlockSpec((200, 1), lambda g: (g, 0)),
            pl.BlockSpec((1, H), lambda g: (0, 0)),
            pl.BlockSpec((1, H), lambda g: (0, 0)),
            pl.BlockSpec((1, H), lambda g: (0, 0)),
            pl.BlockSpec((200, 1), lambda g: (g, 0)),
        ],
        out_specs=[
            pl.BlockSpec((G, H), lambda g: (0, 0)),
            pl.BlockSpec((G, 1), lambda g: (0, 0)),
        ],
        out_shape=[
            jax.ShapeDtypeStruct((G, H), f32),
            jax.ShapeDtypeStruct((G, 1), f32),
        ],
    )(aggp, y, dinv_col, b_row, a_row, d_row, batch_col)


def _final_body(pooled_ref, cnt_ref, wl_ref, bl_ref, out_ref):
    mean = pooled_ref[...] / jnp.maximum(cnt_ref[...], 1.0)
    out_ref[...] = jnp.dot(
        mean, wl_ref[...], preferred_element_type=f32) + bl_ref[...]


def _tfinal(pooled, cnt, Wl, bl_row):
    return pl.pallas_call(
        _final_body,
        out_shape=jax.ShapeDtypeStruct((G, 2), f32),
    )(pooled, cnt, Wl, bl_row)


def kernel(x, edge_index, batch, W1, b1, g1, be1, W2, b2, g2, be2,
           W3, b3, g3, be3, Wl, bl):
    eps = 1e-5
    pad = EP - E
    srcp = jnp.concatenate([edge_index[0], jnp.zeros((pad,), i32)])
    dstp = jnp.concatenate([edge_index[1], jnp.full((pad,), N, i32)])
    packed = jnp.bitwise_or(lax.shift_left(srcp, 16), dstp)
    packed2d = packed.reshape(EROWS, 128)
    zeros_s = jnp.zeros((SUB_S,), f32)
    zeros_f = jnp.zeros((SUB_F, HH), f32)
    xp = jnp.concatenate([x[:, 0], jnp.zeros((NPS - N,), f32)]).reshape(256, 200)

    degp = _deg_kernel(packed, zeros_s).reshape(NC, 256, 200)
    dinv, y1 = _t1(degp, xp)

    aggp1 = _sagg_kernel(packed, y1.reshape(NPS), zeros_s)
    aggp1 = aggp1.reshape(NC, 256, 200)
    s2d, ssum, ssq = _t2a(aggp1, y1, dinv)
    sbar = ssum[0, 0] / N
    var_s = ssq[0, 0] / N - sbar * sbar
    w1 = W1[0]
    a1c = (w1 * g1 / jnp.sqrt(var_s * w1 * w1 + eps)).reshape(1, H)
    d1c = be1.reshape(1, H) - sbar * a1c

    s_col = s2d.reshape(NPS, 1)[:N]
    dinv_col = dinv.reshape(NPS, 1)[:N]
    y2 = _t2b(s_col, dinv_col, a1c, d1c, W2)

    aggp2 = _ragg_kernel(y2.reshape(NC * N, HH), packed2d, zeros_f)
    b2r = b2.reshape(1, H)
    ssum2, ssq2 = _tstats(aggp2, y2, dinv_col, b2r)
    mu2 = ssum2 / N
    var2 = ssq2 / N - mu2 * mu2
    a2c = g2.reshape(1, H) / jnp.sqrt(var2 + eps)
    d2c = be2.reshape(1, H) - mu2 * a2c
    y3 = _tnext(aggp2, y2, dinv_col, b2r, a2c, d2c, W3)

    aggp3 = _ragg_kernel(y3.reshape(NC * N, HH), packed2d, zeros_f)
    b3r = b3.reshape(1, H)
    ssum3, ssq3 = _tstats(aggp3, y3, dinv_col, b3r)
    mu3 = ssum3 / N
    var3 = ssq3 / N - mu3 * mu3
    a3c = g3.reshape(1, H) / jnp.sqrt(var3 + eps)
    d3c = be3.reshape(1, H) - mu3 * a3c

    batch_col = batch.reshape(N, 1)
    pooled, cnt = _tpool(aggp3, y3, dinv_col, b3r, a3c, d3c, batch_col)
    return _tfinal(pooled, cnt, Wl, bl.reshape(1, 2))
